# baseline (speedup 1.0000x reference)
import jax
import jax.numpy as jnp
import numpy as np
from jax import lax
from jax.experimental import pallas as pl
from jax.experimental.pallas import tpu as pltpu
from jax.experimental.pallas import tpu_sc as plsc

N = 10000
E = 320000
D = 128
H = 8
F = 16
ROW = 144
DROW = 16
FFN = 256
OUT = 512

NC = 2
NS = 16
NW = NC * NS
E_PER_W = E // NW
SUB = 80
EROWS = E // SUB
EROWS_PER_W = E_PER_W // SUB
STAGE = 80
NSTAGE = N // STAGE
STAGE_T = -(-NSTAGE // NS)


def _prep_body(x_ref, wg_ref, ab_ref, ts_ref, td_ref):
    x = x_ref[...]
    wh = jnp.dot(x, wg_ref[...], preferred_element_type=jnp.float32)
    al = jnp.dot(wh, ab_ref[...], preferred_element_type=jnp.float32)
    b = x.shape[0]
    z8 = jnp.zeros((b, 8), jnp.float32)
    ts_ref[...] = jnp.concatenate([wh, al[:, :8], z8], axis=1)
    td_ref[...] = jnp.concatenate([al[:, 8:16], z8], axis=1)


def _prep(batch, W_gat, a_both):
    blk = 1000
    grid = N // blk
    return pl.pallas_call(
        _prep_body,
        grid=(grid,),
        in_specs=[
            pl.BlockSpec((blk, D), lambda i: (i, 0)),
            pl.BlockSpec((D, D), lambda i: (0, 0)),
            pl.BlockSpec((D, 16), lambda i: (0, 0)),
        ],
        out_specs=[
            pl.BlockSpec((blk, ROW), lambda i: (i, 0)),
            pl.BlockSpec((blk, DROW), lambda i: (i, 0)),
        ],
        out_shape=[
            jax.ShapeDtypeStruct((N, ROW), jnp.float32),
            jax.ShapeDtypeStruct((N, DROW), jnp.float32),
        ],
    )(batch, W_gat, a_both)


def _bcast_lane(vec, h):
    return jnp.take_along_axis(vec, jnp.full((16,), h, jnp.int32), axis=0)


def _sc_edge_body(ep_hbm, ts_hbm, td_hbm, out_hbm,
                  acc_sh, idx2, sidx, rows3, drows3,
                  sem_g, sem_d, sem_s, sem_i):
    c = lax.axis_index("c")
    s = lax.axis_index("s")
    wid = c * NS + s
    wbase = wid * EROWS_PER_W
    stage = rows3.at[0]

    z16 = jnp.zeros((16,), jnp.float32)

    def zrow(r, _):
        for k in range(ROW // 16):
            rows3[0, r, pl.ds(k * 16, 16)] = z16
        return 0

    lax.fori_loop(0, STAGE, zrow, 0)
    for t in range(STAGE_T):
        k = s + NS * t

        @pl.when(k < NSTAGE)
        def _():
            pltpu.sync_copy(stage, acc_sh.at[pl.ds(k * STAGE, STAGE)])
    plsc.subcore_barrier()

    def fire_idx(t2, m):
        pltpu.async_copy(ep_hbm.at[wbase + t2], idx2.at[m], sem_i)

    def wait_idx(m):
        pltpu.make_async_copy(ep_hbm.at[wbase], idx2.at[m], sem_i).wait()

    def fire_gathers(p, m):
        pltpu.async_copy(ts_hbm.at[idx2.at[m, 0]], rows3.at[p], sem_g)
        pltpu.async_copy(td_hbm.at[idx2.at[m, 1]], drows3.at[p], sem_d)

    def wait_gathers(p, m):
        pltpu.make_async_copy(ts_hbm.at[idx2.at[m, 0]], rows3.at[p],
                              sem_g).wait()
        pltpu.make_async_copy(td_hbm.at[idx2.at[m, 1]], drows3.at[p],
                              sem_d).wait()

    def stash_dst(p, m):
        for k in range(SUB // 16):
            sidx[p, pl.ds(k * 16, 16)] = idx2[m, 1, pl.ds(k * 16, 16)]

    def fire_scatter(p):
        pltpu.async_copy(rows3.at[p], acc_sh.at[sidx.at[p]], sem_s, add=True)

    def wait_scatter(p):
        pltpu.make_async_copy(rows3.at[p], acc_sh.at[sidx.at[p]],
                              sem_s).wait()

    def compute(p):
        def edge(e, _):
            t = rows3[p, e, pl.ds(D, 16)] + drows3[p, e, :]
            ex = jnp.exp(jnp.maximum(t, 0.2 * t))
            rows3[p, e, pl.ds(D, 16)] = ex
            for h in range(H):
                exb = _bcast_lane(ex, h)
                rows3[p, e, pl.ds(h * 16, 16)] = (
                    rows3[p, e, pl.ds(h * 16, 16)] * exb)
            return 0

        lax.fori_loop(0, SUB, edge, 0, unroll=2)

    def step(t, k, wait_prev, fire_ahead):
        p, m = k % 3, k % 2
        wait_gathers(p, m)
        stash_dst(p, m)
        if fire_ahead:
            fire_idx(t + 2, m)
        compute(p)
        fire_scatter(p)
        wait_prev()
        if fire_ahead:
            wait_idx(m)
            fire_gathers((k + 2) % 3, m)

    pltpu.sync_copy(ep_hbm.at[wbase], idx2.at[0])
    fire_gathers(0, 0)
    pltpu.sync_copy(ep_hbm.at[wbase + 1], idx2.at[1])
    fire_gathers(1, 1)

    NB = EROWS_PER_W // 6
    NEPI = EROWS_PER_W - NB * 6

    def body(r, _):
        t0 = 6 * r
        for k in range(6):
            def wp(k=k):
                if k == 0:
                    @pl.when(r > 0)
                    def _():
                        wait_scatter((k - 1) % 3)
                else:
                    wait_scatter((k - 1) % 3)
            step(t0 + k, k, wp, True)
        return 0

    lax.fori_loop(0, NB, body, 0)
    for k in range(NEPI):
        t = NB * 6 + k
        step(t, k, lambda k=k: wait_scatter((k - 1) % 3),
             fire_ahead=(t + 2 < EROWS_PER_W))
    wait_scatter((EROWS_PER_W - 1) % 3)
    plsc.subcore_barrier()

    for t in range(STAGE_T):
        k = s + NS * t

        @pl.when(k < NSTAGE)
        def _():
            pltpu.sync_copy(acc_sh.at[pl.ds(k * STAGE, STAGE)], stage)
            pltpu.sync_copy(stage, out_hbm.at[c, pl.ds(k * STAGE, STAGE)])


def _sc_edge(src, dst, table_src, table_dst):
    mesh = plsc.VectorSubcoreMesh(core_axis_name="c", subcore_axis_name="s")
    fn = pl.kernel(
        _sc_edge_body,
        out_type=jax.ShapeDtypeStruct((NC, N, ROW), jnp.float32),
        mesh=mesh,
        compiler_params=pltpu.CompilerParams(use_tc_tiling_on_sc=False),
        scratch_types=[
            pltpu.VMEM_SHARED((N, ROW), jnp.float32),
            pltpu.VMEM((2, 2, SUB), jnp.int32),
            pltpu.VMEM((3, SUB), jnp.int32),
            pltpu.VMEM((3, SUB, ROW), jnp.float32),
            pltpu.VMEM((3, SUB, DROW), jnp.float32),
            pltpu.SemaphoreType.DMA,
            pltpu.SemaphoreType.DMA,
            pltpu.SemaphoreType.DMA,
            pltpu.SemaphoreType.DMA,
        ],
    )
    ep = jnp.stack([src.reshape(EROWS, SUB), dst.reshape(EROWS, SUB)], axis=1)
    return fn(ep, table_src, table_dst)


_NBLK = 10


def _head_body(a0_ref, a1_ref, x_ref, wf_ref, bf_ref, wh_ref, bh_ref,
               wo_ref, bo_ref, wd_ref, sig_ref, loss_ref,
               csum_ref, proj_ref):
    i = pl.program_id(0)

    @pl.when(i < _NBLK)
    def _():
        a = a0_ref[...] + a1_ref[...]
        b = a.shape[0]
        num = a[:, :D]
        den = a[:, D:D + H]
        denb = jnp.broadcast_to(den.reshape(b, H, 1), (b, H, F)).reshape(b, D)
        emb = num / (denb + 1e-16)
        emb = jnp.where(emb > 0, emb, jnp.exp(jnp.minimum(emb, 0.0)) - 1.0)
        h1 = jnp.maximum(
            jnp.dot(emb, wf_ref[...], preferred_element_type=jnp.float32)
            + bf_ref[...], 0.0)
        h2 = jnp.maximum(
            jnp.dot(h1, wh_ref[...], preferred_element_type=jnp.float32)
            + bh_ref[...], 0.0)
        lg = (jnp.dot(h2, wo_ref[...], preferred_element_type=jnp.float32)
              + bo_ref[...])
        sig_ref[...] = jax.nn.sigmoid(lg)
        csum = jnp.sum(emb, axis=0, keepdims=True)

        @pl.when(i == 0)
        def _():
            csum_ref[...] = csum

        @pl.when(i > 0)
        def _():
            csum_ref[...] = csum_ref[...] + csum

    @pl.when(i >= _NBLK)
    def _():
        @pl.when(i == _NBLK)
        def _():
            summary = jax.nn.sigmoid(csum_ref[...] / float(N))
            pr = jnp.sum(wd_ref[...] * summary, axis=1)
            proj_ref[...] = pr.reshape(1, D)

        x = x_ref[...]
        p = jnp.sum(x * proj_ref[...], axis=1, keepdims=True)
        ls_p = jnp.minimum(p, 0.0) - jnp.log1p(jnp.exp(-jnp.abs(p)))
        ls_n = jnp.minimum(-p, 0.0) - jnp.log1p(jnp.exp(-jnp.abs(p)))
        val = jnp.reshape(-(jnp.sum(ls_p) + jnp.sum(ls_n)) / float(N), (1, 1))

        @pl.when(i == _NBLK)
        def _():
            loss_ref[...] = val

        @pl.when(i > _NBLK)
        def _():
            loss_ref[...] = loss_ref[...] + val


def _head(acc0, acc1, batch, W_fus, b_fus, W_h, b_h, W_out, b_out, W_dgi):
    blk = N // _NBLK
    return pl.pallas_call(
        _head_body,
        grid=(2 * _NBLK,),
        in_specs=[
            pl.BlockSpec((blk, ROW), lambda i: (jnp.minimum(i, _NBLK - 1), 0)),
            pl.BlockSpec((blk, ROW), lambda i: (jnp.minimum(i, _NBLK - 1), 0)),
            pl.BlockSpec((blk, D), lambda i: (jnp.maximum(i - _NBLK, 0), 0)),
            pl.BlockSpec((D, FFN), lambda i: (0, 0)),
            pl.BlockSpec((1, FFN), lambda i: (0, 0)),
            pl.BlockSpec((FFN, FFN), lambda i: (0, 0)),
            pl.BlockSpec((1, FFN), lambda i: (0, 0)),
            pl.BlockSpec((FFN, OUT), lambda i: (0, 0)),
            pl.BlockSpec((1, OUT), lambda i: (0, 0)),
            pl.BlockSpec((D, D), lambda i: (0, 0)),
        ],
        out_specs=[
            pl.BlockSpec((blk, OUT), lambda i: (jnp.minimum(i, _NBLK - 1), 0)),
            pl.BlockSpec((1, 1), lambda i: (0, 0)),
        ],
        out_shape=[
            jax.ShapeDtypeStruct((N, OUT), jnp.float32),
            jax.ShapeDtypeStruct((1, 1), jnp.float32),
        ],
        scratch_shapes=[
            pltpu.VMEM((1, D), jnp.float32),
            pltpu.VMEM((1, D), jnp.float32),
        ],
    )(acc0, acc1, batch, W_fus, b_fus, W_h, b_h, W_out, b_out, W_dgi)


def kernel(batch, edge_index, drug_nums, W_gat, a_src, a_dst, W_fus, b_fus,
           W_h, b_h, W_out, b_out, W_dgi):
    src = edge_index[0]
    dst = edge_index[1]
    eye = jnp.eye(H, dtype=jnp.float32)
    a_s = (eye[:, None, :] * a_src[:, :, None]).reshape(D, H)
    a_d = (eye[:, None, :] * a_dst[:, :, None]).reshape(D, H)
    a_both = jnp.concatenate([a_s, a_d], axis=1)

    table_src, table_dst = _prep(batch, W_gat, a_both)
    acc = _sc_edge(src, dst, table_src, table_dst)
    sig, loss = _head(acc[0], acc[1], batch, W_fus, b_fus.reshape(1, FFN),
                      W_h, b_h.reshape(1, FFN), W_out, b_out.reshape(1, OUT),
                      W_dgi)
    return (sig.reshape(-1), loss.reshape(()))


# speedup vs baseline: 1.2494x; 1.2494x over previous
import jax
import jax.numpy as jnp
import numpy as np
from jax import lax
from jax.experimental import pallas as pl
from jax.experimental.pallas import tpu as pltpu
from jax.experimental.pallas import tpu_sc as plsc

N = 10000
E = 320000
D = 128
H = 8
F = 16
ROW = 144
DROW = 16
FFN = 256
OUT = 512

NC = 2
NS = 16
NW = NC * NS
E_PER_W = E // NW
SUB = 80
EROWS = E // SUB
EROWS_PER_W = E_PER_W // SUB
STAGE = 80
NSTAGE = N // STAGE
STAGE_T = -(-NSTAGE // NS)


def _prep_body(x_ref, wg_ref, ab_ref, ts_ref, td_ref):
    x = x_ref[...]
    wh = jnp.dot(x, wg_ref[...], preferred_element_type=jnp.float32)
    al = jnp.dot(wh, ab_ref[...], preferred_element_type=jnp.float32)
    b = x.shape[0]
    z8 = jnp.zeros((b, 8), jnp.float32)
    ts_ref[...] = jnp.concatenate([wh, al[:, :8], z8], axis=1)
    td_ref[...] = jnp.concatenate([al[:, 8:16], z8], axis=1)


def _prep(batch, W_gat, a_both):
    blk = 1000
    grid = N // blk
    return pl.pallas_call(
        _prep_body,
        grid=(grid,),
        in_specs=[
            pl.BlockSpec((blk, D), lambda i: (i, 0)),
            pl.BlockSpec((D, D), lambda i: (0, 0)),
            pl.BlockSpec((D, 16), lambda i: (0, 0)),
        ],
        out_specs=[
            pl.BlockSpec((blk, ROW), lambda i: (i, 0)),
            pl.BlockSpec((blk, DROW), lambda i: (i, 0)),
        ],
        out_shape=[
            jax.ShapeDtypeStruct((N, ROW), jnp.float32),
            jax.ShapeDtypeStruct((N, DROW), jnp.float32),
        ],
    )(batch, W_gat, a_both)


def _bcast_lane(vec, h):
    return jnp.take_along_axis(vec, jnp.full((16,), h, jnp.int32), axis=0)


def _sc_edge_body(ep_hbm, ts_hbm, td_hbm, out_hbm,
                  acc_sh, idx2, sidx, rows3, drows3,
                  sem_g, sem_d, sem_s, sem_i):
    c = lax.axis_index("c")
    s = lax.axis_index("s")
    wid = c * NS + s
    wbase = wid * EROWS_PER_W
    stage = rows3.at[0]

    z16 = jnp.zeros((16,), jnp.float32)

    def zrow(r, _):
        for k in range(ROW // 16):
            rows3[0, r, pl.ds(k * 16, 16)] = z16
        return 0

    lax.fori_loop(0, STAGE, zrow, 0)
    for t in range(STAGE_T):
        k = s + NS * t

        @pl.when(k < NSTAGE)
        def _():
            pltpu.sync_copy(stage, acc_sh.at[pl.ds(k * STAGE, STAGE)])
    plsc.subcore_barrier()

    def fire_idx(t2, m):
        pltpu.async_copy(ep_hbm.at[wbase + t2], idx2.at[m], sem_i)

    def wait_idx(m):
        pltpu.make_async_copy(ep_hbm.at[wbase], idx2.at[m], sem_i).wait()

    def fire_gathers(p, m):
        pltpu.async_copy(ts_hbm.at[idx2.at[m, 0]], rows3.at[p], sem_g)
        pltpu.async_copy(td_hbm.at[idx2.at[m, 1]], drows3.at[p], sem_d)

    def wait_gathers(p, m):
        pltpu.make_async_copy(ts_hbm.at[idx2.at[m, 0]], rows3.at[p],
                              sem_g).wait()
        pltpu.make_async_copy(td_hbm.at[idx2.at[m, 1]], drows3.at[p],
                              sem_d).wait()

    def stash_dst(p, m):
        for k in range(SUB // 16):
            sidx[p, pl.ds(k * 16, 16)] = idx2[m, 1, pl.ds(k * 16, 16)]

    def fire_scatter(p):
        pltpu.async_copy(rows3.at[p], acc_sh.at[sidx.at[p]], sem_s, add=True)

    def wait_scatter(p):
        pltpu.make_async_copy(rows3.at[p], acc_sh.at[sidx.at[p]],
                              sem_s).wait()

    def compute(p):
        @plsc.parallel_loop(0, SUB, 1, unroll=2)
        def edge(e):
            t = rows3[p, e, pl.ds(D, 16)] + drows3[p, e, :]
            ex = jnp.exp(jnp.maximum(t, 0.2 * t))
            rows3[p, e, pl.ds(D, 16)] = ex
            for h in range(H):
                exb = _bcast_lane(ex, h)
                rows3[p, e, pl.ds(h * 16, 16)] = (
                    rows3[p, e, pl.ds(h * 16, 16)] * exb)

    def step(t, k, wait_prev, fire_ahead):
        p, m = k % 3, k % 2
        wait_gathers(p, m)
        stash_dst(p, m)
        if fire_ahead:
            fire_idx(t + 2, m)
        compute(p)
        fire_scatter(p)
        wait_prev()
        if fire_ahead:
            wait_idx(m)
            fire_gathers((k + 2) % 3, m)

    pltpu.sync_copy(ep_hbm.at[wbase], idx2.at[0])
    fire_gathers(0, 0)
    pltpu.sync_copy(ep_hbm.at[wbase + 1], idx2.at[1])
    fire_gathers(1, 1)

    NB = EROWS_PER_W // 6
    NEPI = EROWS_PER_W - NB * 6

    def body(r, _):
        t0 = 6 * r
        for k in range(6):
            def wp(k=k):
                if k == 0:
                    @pl.when(r > 0)
                    def _():
                        wait_scatter((k - 1) % 3)
                else:
                    wait_scatter((k - 1) % 3)
            step(t0 + k, k, wp, True)
        return 0

    lax.fori_loop(0, NB, body, 0)
    for k in range(NEPI):
        t = NB * 6 + k
        step(t, k, lambda k=k: wait_scatter((k - 1) % 3),
             fire_ahead=(t + 2 < EROWS_PER_W))
    wait_scatter((EROWS_PER_W - 1) % 3)
    plsc.subcore_barrier()

    for t in range(STAGE_T):
        k = s + NS * t

        @pl.when(k < NSTAGE)
        def _():
            pltpu.sync_copy(acc_sh.at[pl.ds(k * STAGE, STAGE)],
                            out_hbm.at[c, pl.ds(k * STAGE, STAGE)])


def _sc_edge(src, dst, table_src, table_dst):
    mesh = plsc.VectorSubcoreMesh(core_axis_name="c", subcore_axis_name="s")
    fn = pl.kernel(
        _sc_edge_body,
        out_type=jax.ShapeDtypeStruct((NC, N, ROW), jnp.float32),
        mesh=mesh,
        compiler_params=pltpu.CompilerParams(use_tc_tiling_on_sc=False),
        scratch_types=[
            pltpu.VMEM_SHARED((N, ROW), jnp.float32),
            pltpu.VMEM((2, 2, SUB), jnp.int32),
            pltpu.VMEM((3, SUB), jnp.int32),
            pltpu.VMEM((3, SUB, ROW), jnp.float32),
            pltpu.VMEM((3, SUB, DROW), jnp.float32),
            pltpu.SemaphoreType.DMA,
            pltpu.SemaphoreType.DMA,
            pltpu.SemaphoreType.DMA,
            pltpu.SemaphoreType.DMA,
        ],
    )
    ep = jnp.stack([src.reshape(EROWS, SUB), dst.reshape(EROWS, SUB)], axis=1)
    return fn(ep, table_src, table_dst)


def _head_body(a0_ref, a1_ref, wf_ref, bf_ref, wh_ref, bh_ref,
               wo_ref, bo_ref, sig_ref, csum_ref):
    i = pl.program_id(0)
    a = a0_ref[...] + a1_ref[...]
    b = a.shape[0]
    num = a[:, :D]
    den = a[:, D:D + H]
    denb = jnp.broadcast_to(den.reshape(b, H, 1), (b, H, F)).reshape(b, D)
    emb = num / (denb + 1e-16)
    emb = jnp.where(emb > 0, emb, jnp.exp(jnp.minimum(emb, 0.0)) - 1.0)
    h1 = jnp.maximum(
        jnp.dot(emb, wf_ref[...], preferred_element_type=jnp.float32)
        + bf_ref[...], 0.0)
    h2 = jnp.maximum(
        jnp.dot(h1, wh_ref[...], preferred_element_type=jnp.float32)
        + bh_ref[...], 0.0)
    lg = (jnp.dot(h2, wo_ref[...], preferred_element_type=jnp.float32)
          + bo_ref[...])
    sig_ref[...] = jax.nn.sigmoid(lg)
    csum = jnp.sum(emb, axis=0, keepdims=True)

    @pl.when(i == 0)
    def _():
        csum_ref[...] = csum

    @pl.when(i > 0)
    def _():
        csum_ref[...] = csum_ref[...] + csum


def _head(acc0, acc1, W_fus, b_fus, W_h, b_h, W_out, b_out):
    blk = 1000
    grid = N // blk
    return pl.pallas_call(
        _head_body,
        grid=(grid,),
        in_specs=[
            pl.BlockSpec((blk, ROW), lambda i: (i, 0)),
            pl.BlockSpec((blk, ROW), lambda i: (i, 0)),
            pl.BlockSpec((D, FFN), lambda i: (0, 0)),
            pl.BlockSpec((1, FFN), lambda i: (0, 0)),
            pl.BlockSpec((FFN, FFN), lambda i: (0, 0)),
            pl.BlockSpec((1, FFN), lambda i: (0, 0)),
            pl.BlockSpec((FFN, OUT), lambda i: (0, 0)),
            pl.BlockSpec((1, OUT), lambda i: (0, 0)),
        ],
        out_specs=[
            pl.BlockSpec((blk, OUT), lambda i: (i, 0)),
            pl.BlockSpec((1, D), lambda i: (0, 0)),
        ],
        out_shape=[
            jax.ShapeDtypeStruct((N, OUT), jnp.float32),
            jax.ShapeDtypeStruct((1, D), jnp.float32),
        ],
    )(acc0, acc1, W_fus, b_fus, W_h, b_h, W_out, b_out)


def _dgi_body(x_ref, cs_ref, wd_ref, loss_ref, proj_ref):
    i = pl.program_id(0)

    @pl.when(i == 0)
    def _():
        summary = jax.nn.sigmoid(cs_ref[...] / float(N))
        pr = jnp.sum(wd_ref[...] * summary, axis=1)
        proj_ref[...] = pr.reshape(1, D)

    x = x_ref[...]
    p = jnp.sum(x * proj_ref[...], axis=1, keepdims=True)
    ls_p = jnp.minimum(p, 0.0) - jnp.log1p(jnp.exp(-jnp.abs(p)))
    ls_n = jnp.minimum(-p, 0.0) - jnp.log1p(jnp.exp(-jnp.abs(p)))
    val = jnp.reshape(-(jnp.sum(ls_p) + jnp.sum(ls_n)) / float(N), (1, 1))

    @pl.when(i == 0)
    def _():
        loss_ref[...] = val

    @pl.when(i > 0)
    def _():
        loss_ref[...] = loss_ref[...] + val


def _dgi(batch, csum, W_dgi):
    blk = 1000
    grid = N // blk
    return pl.pallas_call(
        _dgi_body,
        grid=(grid,),
        in_specs=[
            pl.BlockSpec((blk, D), lambda i: (i, 0)),
            pl.BlockSpec((1, D), lambda i: (0, 0)),
            pl.BlockSpec((D, D), lambda i: (0, 0)),
        ],
        out_specs=pl.BlockSpec((1, 1), lambda i: (0, 0)),
        out_shape=jax.ShapeDtypeStruct((1, 1), jnp.float32),
        scratch_shapes=[pltpu.VMEM((1, D), jnp.float32)],
    )(batch, csum, W_dgi)


def kernel(batch, edge_index, drug_nums, W_gat, a_src, a_dst, W_fus, b_fus,
           W_h, b_h, W_out, b_out, W_dgi):
    src = edge_index[0]
    dst = edge_index[1]
    eye = jnp.eye(H, dtype=jnp.float32)
    a_s = (eye[:, None, :] * a_src[:, :, None]).reshape(D, H)
    a_d = (eye[:, None, :] * a_dst[:, :, None]).reshape(D, H)
    a_both = jnp.concatenate([a_s, a_d], axis=1)

    table_src, table_dst = _prep(batch, W_gat, a_both)
    acc = _sc_edge(src, dst, table_src, table_dst)
    sig, csum = _head(acc[0], acc[1], W_fus, b_fus.reshape(1, FFN),
                      W_h, b_h.reshape(1, FFN), W_out, b_out.reshape(1, OUT))
    loss = _dgi(batch, csum, W_dgi)
    return (sig.reshape(-1), loss.reshape(()))
